# baseline (speedup 1.0000x reference)
import functools

import jax
import jax.numpy as jnp
from jax import lax
from jax.experimental import pallas as pl
from jax.experimental.pallas import tpu as pltpu
from jax.experimental.pallas import tpu_sc as plsc

N = 10000
E = 320000
D = 128
NC = 2
NS = 16
NW = NC * NS
B_PER_W = E // NW
C = 80
N_CHUNKS = B_PER_W // C
PS_STRIDE = 17


def _rne_bf16_bits(bits):
    return (bits + 0x7FFF + ((bits >> 16) & 1)) >> 16


def _pack_pairs(x_f32):
    bits = jax.lax.bitcast_convert_type(x_f32, jnp.int32)
    h = x_f32.shape[-1] // 2
    lo = _rne_bf16_bits(bits[:, :h]) & 0xFFFF
    hi = _rne_bf16_bits(bits[:, h:]) << 16
    return lo | hi


def _zm_body(z_ref, m_ref, zm_ref, zb_ref):
    zm = jnp.dot(z_ref[...], m_ref[...], preferred_element_type=jnp.float32)
    zm_ref[...] = _pack_pairs(zm)
    zb_ref[...] = _pack_pairs(z_ref[...])


def _compute_zm(z, M):
    return pl.pallas_call(
        _zm_body,
        out_shape=[jax.ShapeDtypeStruct((N, D // 2), jnp.int32),
                   jax.ShapeDtypeStruct((N, D // 2), jnp.int32)],
    )(z, M)


@functools.cache
def _build_sc_edge_dot():
    mesh = plsc.VectorSubcoreMesh(core_axis_name="c", subcore_axis_name="s",
                                  num_cores=NC, num_subcores=NS)

    @functools.partial(
        pl.kernel,
        out_type=jax.ShapeDtypeStruct((E,), jnp.float32),
        mesh=mesh,
        compiler_params=pltpu.CompilerParams(needs_layout_passes=False,
                                             use_tc_tiling_on_sc=False),
        scratch_types=[
            pltpu.VMEM((B_PER_W,), jnp.int32),
            pltpu.VMEM((B_PER_W,), jnp.int32),
            pltpu.VMEM((2, C, D // 2), jnp.int32),
            pltpu.VMEM((2, C, D // 2), jnp.int32),
            pltpu.VMEM((C * PS_STRIDE,), jnp.float32),
            pltpu.VMEM((2, C), jnp.float32),
            pltpu.VMEM_SHARED((N, D // 2), jnp.int32),
            pltpu.VMEM_SHARED((N, D // 2), jnp.int32),
            pltpu.SemaphoreType.DMA,
            pltpu.SemaphoreType.DMA,
            pltpu.SemaphoreType.DMA,
            pltpu.SemaphoreType.DMA,
            pltpu.SemaphoreType.DMA,
            pltpu.SemaphoreType.DMA,
        ],
    )
    def _sc_edge_dot(zm_hbm, z_hbm, src_hbm, dst_hbm, out_hbm,
                     si, di, sr, dr, ps, ob, zm_sp, z_sp,
                     sem_s0, sem_s1, sem_d0, sem_d1, sem_o0, sem_o1):
        sid = lax.axis_index("s")
        wid = sid * NC + lax.axis_index("c")
        base = wid * B_PER_W
        lanes = lax.iota(jnp.int32, 16)
        sems = ((sem_s0, sem_d0), (sem_s1, sem_d1))
        osems = (sem_o0, sem_o1)

        rows_per_tile = N // NS
        stg = sid * rows_per_tile
        pltpu.sync_copy(zm_hbm.at[pl.ds(stg, rows_per_tile)],
                        zm_sp.at[pl.ds(stg, rows_per_tile)])
        pltpu.sync_copy(z_hbm.at[pl.ds(stg, rows_per_tile)],
                        z_sp.at[pl.ds(stg, rows_per_tile)])
        pltpu.sync_copy(src_hbm.at[pl.ds(base, B_PER_W)], si)
        pltpu.sync_copy(dst_hbm.at[pl.ds(base, B_PER_W)], di)
        plsc.subcore_barrier()

        def start_gather(off, b, n_rows):
            idx_s = si.at[pl.ds(off, n_rows)]
            idx_d = di.at[pl.ds(off, n_rows)]
            pltpu.make_async_copy(zm_sp.at[idx_s],
                                  sr.at[b, pl.ds(0, n_rows)], sems[b][0]).start()
            pltpu.make_async_copy(z_sp.at[idx_d],
                                  dr.at[b, pl.ds(0, n_rows)], sems[b][1]).start()

        def wait_gather(b, n_rows):
            pltpu.make_async_copy(zm_sp.at[si.at[pl.ds(0, n_rows)]],
                                  sr.at[b, pl.ds(0, n_rows)], sems[b][0]).wait()
            pltpu.make_async_copy(z_sp.at[di.at[pl.ds(0, n_rows)]],
                                  dr.at[b, pl.ds(0, n_rows)], sems[b][1]).wait()

        def compute(b, n_groups):
            srb = sr.at[b]
            drb = dr.at[b]

            @plsc.parallel_loop(0, n_groups * 16, unroll=4)
            def edge_body(e):
                prods = []
                for j in range(D // 32):
                    sv = plsc.bitcast(srb[e, pl.ds(j * 16, 16)],
                                      jnp.bfloat16)
                    dv = plsc.bitcast(drb[e, pl.ds(j * 16, 16)],
                                      jnp.bfloat16)
                    prods.append(sv * dv)
                t = (prods[0] + prods[1]) + (prods[2] + prods[3])
                pa, pb = plsc.unpack(
                    t, format=plsc.PackFormat.INTERLEAVED,
                    preferred_element_type=jnp.float32)
                plsc.store_scatter(ps, [e * PS_STRIDE + lanes], pa + pb)

            for g in range(n_groups):
                gl = g * 16 + lanes
                outv = plsc.load_gather(ps, [gl * PS_STRIDE])
                for l in range(1, 16):
                    outv = outv + plsc.load_gather(ps, [gl * PS_STRIDE + l])
                ob[b, pl.ds(g * 16, 16)] = outv

        def start_out(off, b, n_rows):
            pltpu.make_async_copy(ob.at[b, pl.ds(0, n_rows)],
                                  out_hbm.at[pl.ds(base + off, n_rows)],
                                  osems[b]).start()

        def wait_out(b, n_rows):
            pltpu.make_async_copy(ob.at[b, pl.ds(0, n_rows)],
                                  out_hbm.at[pl.ds(base, n_rows)],
                                  osems[b]).wait()

        start_gather(0, 0, C)

        def pair_body(p, carry):
            wait_gather(0, C)
            start_gather((2 * p + 1) * C, 1, C)

            @pl.when(p > 0)
            def _():
                wait_out(0, C)

            compute(0, C // 16)
            start_out(2 * p * C, 0, C)
            wait_gather(1, C)
            start_gather((2 * p + 2) * C, 0, C)

            @pl.when(p > 0)
            def _():
                wait_out(1, C)

            compute(1, C // 16)
            start_out((2 * p + 1) * C, 1, C)
            return carry

        lax.fori_loop(0, N_CHUNKS // 2, pair_body, 0)
        wait_gather(0, C)
        wait_out(0, C)
        compute(0, C // 16)
        start_out((N_CHUNKS - 1) * C, 0, C)
        wait_out(0, C)
        wait_out(1, C)

    return _sc_edge_dot


def kernel(z, edge_index, M):
    zm_i, z_i = _compute_zm(z, M)
    src = edge_index[0].astype(jnp.int32)
    dst = edge_index[1].astype(jnp.int32)
    return _build_sc_edge_dot()(zm_i, z_i, src, dst)


# speedup vs baseline: 1.0855x; 1.0855x over previous
import functools

import jax
import jax.numpy as jnp
from jax import lax
from jax.experimental import pallas as pl
from jax.experimental.pallas import tpu as pltpu
from jax.experimental.pallas import tpu_sc as plsc

N = 10000
E = 320000
D = 128
NC = 2
NS = 16
NW = NC * NS
B_PER_W = E // NW
C = 80
N_CHUNKS = B_PER_W // C
PS_STRIDE = 17


def _rne_bf16_bits(bits):
    return (bits + 0x7FFF + ((bits >> 16) & 1)) >> 16


def _pack_pairs(x_f32):
    bits = jax.lax.bitcast_convert_type(x_f32, jnp.int32)
    h = x_f32.shape[-1] // 2
    lo = _rne_bf16_bits(bits[:, :h]) & 0xFFFF
    hi = _rne_bf16_bits(bits[:, h:]) << 16
    return lo | hi


def _zm_body(z_ref, m_ref, zm_ref, zb_ref):
    zm = jnp.dot(z_ref[...], m_ref[...], preferred_element_type=jnp.float32)
    zm_ref[...] = _pack_pairs(zm)
    zb_ref[...] = _pack_pairs(z_ref[...])


def _compute_zm(z, M):
    return pl.pallas_call(
        _zm_body,
        out_shape=[jax.ShapeDtypeStruct((N, D // 2), jnp.int32),
                   jax.ShapeDtypeStruct((N, D // 2), jnp.int32)],
    )(z, M)


@functools.cache
def _build_sc_edge_dot():
    mesh = plsc.VectorSubcoreMesh(core_axis_name="c", subcore_axis_name="s",
                                  num_cores=NC, num_subcores=NS)

    @functools.partial(
        pl.kernel,
        out_type=jax.ShapeDtypeStruct((E,), jnp.float32),
        mesh=mesh,
        compiler_params=pltpu.CompilerParams(needs_layout_passes=False,
                                             use_tc_tiling_on_sc=False),
        scratch_types=[
            pltpu.VMEM((B_PER_W,), jnp.int32),
            pltpu.VMEM((B_PER_W,), jnp.int32),
            pltpu.VMEM((2, C, D // 2), jnp.int32),
            pltpu.VMEM((2, C, D // 2), jnp.int32),
            pltpu.VMEM((C * PS_STRIDE,), jnp.float32),
            pltpu.VMEM((2, C), jnp.float32),
            pltpu.VMEM_SHARED((N, D // 2), jnp.int32),
            pltpu.VMEM_SHARED((N, D // 2), jnp.int32),
            pltpu.SemaphoreType.DMA,
            pltpu.SemaphoreType.DMA,
            pltpu.SemaphoreType.DMA,
            pltpu.SemaphoreType.DMA,
            pltpu.SemaphoreType.DMA,
            pltpu.SemaphoreType.DMA,
        ],
    )
    def _sc_edge_dot(zm_hbm, z_hbm, ei_hbm, out_hbm,
                     si, di, sr, dr, ps, ob, zm_sp, z_sp,
                     sem_s0, sem_s1, sem_d0, sem_d1, sem_o0, sem_o1):
        sid = lax.axis_index("s")
        wid = sid * NC + lax.axis_index("c")
        base = wid * B_PER_W
        lanes = lax.iota(jnp.int32, 16)
        sems = ((sem_s0, sem_d0), (sem_s1, sem_d1))
        osems = (sem_o0, sem_o1)

        rows_per_tile = N // NS
        stg = sid * rows_per_tile
        pltpu.sync_copy(zm_hbm.at[pl.ds(stg, rows_per_tile)],
                        zm_sp.at[pl.ds(stg, rows_per_tile)])
        pltpu.sync_copy(z_hbm.at[pl.ds(stg, rows_per_tile)],
                        z_sp.at[pl.ds(stg, rows_per_tile)])
        pltpu.sync_copy(ei_hbm.at[0, pl.ds(base, B_PER_W)], si)
        pltpu.sync_copy(ei_hbm.at[1, pl.ds(base, B_PER_W)], di)
        plsc.subcore_barrier()

        def start_gather(off, b, n_rows):
            idx_s = si.at[pl.ds(off, n_rows)]
            idx_d = di.at[pl.ds(off, n_rows)]
            pltpu.make_async_copy(zm_sp.at[idx_s],
                                  sr.at[b, pl.ds(0, n_rows)], sems[b][0]).start()
            pltpu.make_async_copy(z_sp.at[idx_d],
                                  dr.at[b, pl.ds(0, n_rows)], sems[b][1]).start()

        def wait_gather(b, n_rows):
            pltpu.make_async_copy(zm_sp.at[si.at[pl.ds(0, n_rows)]],
                                  sr.at[b, pl.ds(0, n_rows)], sems[b][0]).wait()
            pltpu.make_async_copy(z_sp.at[di.at[pl.ds(0, n_rows)]],
                                  dr.at[b, pl.ds(0, n_rows)], sems[b][1]).wait()

        def compute(b, n_groups):
            srb = sr.at[b]
            drb = dr.at[b]

            @plsc.parallel_loop(0, n_groups * 16, unroll=4)
            def edge_body(e):
                prods = []
                for j in range(D // 32):
                    sv = plsc.bitcast(srb[e, pl.ds(j * 16, 16)],
                                      jnp.bfloat16)
                    dv = plsc.bitcast(drb[e, pl.ds(j * 16, 16)],
                                      jnp.bfloat16)
                    prods.append(sv * dv)
                t = (prods[0] + prods[1]) + (prods[2] + prods[3])
                pa, pb = plsc.unpack(
                    t, format=plsc.PackFormat.INTERLEAVED,
                    preferred_element_type=jnp.float32)
                plsc.store_scatter(ps, [e * PS_STRIDE + lanes], pa + pb)

            for g in range(n_groups):
                gl = g * 16 + lanes
                outv = plsc.load_gather(ps, [gl * PS_STRIDE])
                for l in range(1, 16):
                    outv = outv + plsc.load_gather(ps, [gl * PS_STRIDE + l])
                ob[b, pl.ds(g * 16, 16)] = outv

        def start_out(off, b, n_rows):
            pltpu.make_async_copy(ob.at[b, pl.ds(0, n_rows)],
                                  out_hbm.at[pl.ds(base + off, n_rows)],
                                  osems[b]).start()

        def wait_out(b, n_rows):
            pltpu.make_async_copy(ob.at[b, pl.ds(0, n_rows)],
                                  out_hbm.at[pl.ds(base, n_rows)],
                                  osems[b]).wait()

        start_gather(0, 0, C)

        def pair_body(p, carry):
            wait_gather(0, C)
            start_gather((2 * p + 1) * C, 1, C)

            @pl.when(p > 0)
            def _():
                wait_out(0, C)

            compute(0, C // 16)
            start_out(2 * p * C, 0, C)
            wait_gather(1, C)
            start_gather((2 * p + 2) * C, 0, C)

            @pl.when(p > 0)
            def _():
                wait_out(1, C)

            compute(1, C // 16)
            start_out((2 * p + 1) * C, 1, C)
            return carry

        lax.fori_loop(0, N_CHUNKS // 2, pair_body, 0)
        wait_gather(0, C)
        wait_out(0, C)
        compute(0, C // 16)
        start_out((N_CHUNKS - 1) * C, 0, C)
        wait_out(0, C)
        wait_out(1, C)

    return _sc_edge_dot


def kernel(z, edge_index, M):
    zm_i, z_i = _compute_zm(z, M)
    return _build_sc_edge_dot()(zm_i, z_i, edge_index.astype(jnp.int32))
